# baseline (speedup 1.0000x reference)
import jax
import jax.numpy as jnp
from jax import lax
from jax.experimental import pallas as pl
from jax.experimental.pallas import tpu as pltpu
from jax.experimental.pallas import tpu_sc as plsc

NC = 2
NS = 16
L = 16
NW = NC * NS
CHUNK = 128
HID = 128
SEQ = 200
PAD_ID = 0
PAD_ROW = SEQ + CHUNK
PEXT_ROWS = 336


def _build(n_tok):
    per_w = n_tok // NW
    cpw = per_w // CHUNK
    assert cpw % 2 == 0
    mesh = plsc.VectorSubcoreMesh(core_axis_name="c", subcore_axis_name="s")

    def body(enc_ids, dec_ids, src_tab, trg_tab, pext_hbm, enc_out, dec_out,
             idx_big, pext, tok0, tok1, tok2, tok3,
             sem_t0, sem_t1, sem_t2, sem_t3, sem_o0, sem_o1, sem_o2, sem_o3):
        wid = lax.axis_index("s") * NC + lax.axis_index("c")
        tok = (tok0, tok1, tok2, tok3)
        sem_t = (sem_t0, sem_t1, sem_t2, sem_t3)
        sem_o = (sem_o0, sem_o1, sem_o2, sem_o3)
        NBUF = 4

        pltpu.sync_copy(pext_hbm, pext)

        for ids_hbm, tab_hbm, out_hbm in ((enc_ids, src_tab, enc_out),
                                          (dec_ids, trg_tab, dec_out)):
            pltpu.sync_copy(ids_hbm.at[pl.ds(wid * per_w, per_w)], idx_big)

            def issue(m, s):
                pltpu.async_copy(tab_hbm.at[idx_big.at[pl.ds(m * CHUNK, CHUNK)]],
                                 tok[s], sem_t[s])

            def wait_out(m, s):
                base = (wid * cpw + m) * CHUNK
                pltpu.make_async_copy(tok[s], out_hbm.at[pl.ds(base, CHUNK)],
                                      sem_o[s]).wait()

            def consume(m, s, r0):
                pltpu.make_async_copy(tab_hbm.at[idx_big.at[pl.ds(m * CHUNK,
                                                                  CHUNK)]],
                                      tok[s], sem_t[s]).wait()
                base = (wid * cpw + m) * CHUNK

                @plsc.parallel_loop(0, CHUNK, unroll=4)
                def _tok(i):
                    i2 = r0 + i
                    for j in range(HID // L):
                        sl = pl.ds(j * L, L)
                        plsc.addupdate(tok[s].at[i, sl], pext[i2, sl])

                pltpu.async_copy(tok[s], out_hbm.at[pl.ds(base, CHUNK)],
                                 sem_o[s])

            for s in range(NBUF):
                issue(s, s)

            def r0_next(r0):
                r = r0 + CHUNK % SEQ
                return jnp.where(r >= SEQ, r - SEQ, r)

            n_ring = (cpw // NBUF) * NBUF

            @pl.loop(0, n_ring, step=NBUF, init_carry=jnp.int32(0))
            def _chunks(c, r0):
                for s in range(NBUF):
                    consume(c + s, s, r0)
                    r0 = r0_next(r0)
                    sp = (s - 2) % NBUF

                    @pl.when(c + s >= 2)
                    def _(s=s, sp=sp):
                        wait_out(c + s - 2, sp)

                    @pl.when(jnp.logical_and(c + s >= 2, c + s + 2 < cpw))
                    def _(s=s, sp=sp):
                        issue(c + s + 2, sp)
                return r0

            for t in range(n_ring, cpw):
                consume(t, t % NBUF, jnp.int32((t * CHUNK) % SEQ))

            for t in range(cpw - 2 - (cpw - n_ring), cpw):
                wait_out(t, t % NBUF)

            @pl.loop(0, per_w // L, init_carry=jnp.zeros((L,), jnp.int32))
            def _gate(g, acc):
                ids16 = idx_big[pl.ds(g * L, L)]
                return acc + jnp.where(ids16 == PAD_ID, 1, 0)

            macc = _gate
            tot = macc[0]
            for k in range(1, L):
                tot = tot + macc[k]

            @pl.when(tot > 0)
            def _():
                @pl.loop(0, cpw)
                def _chk(m):
                    cacc = None
                    for g in range(CHUNK // L):
                        ids16 = idx_big[pl.ds(m * CHUNK + g * L, L)]
                        pm = jnp.where(ids16 == PAD_ID, 1, 0)
                        cacc = pm if cacc is None else cacc + pm
                    ctot = cacc[0]
                    for k in range(1, L):
                        ctot = ctot + cacc[k]

                    @pl.when(ctot > 0)
                    def _():
                        @pl.loop(0, CHUNK // L)
                        def _grp(g):
                            ids16 = idx_big[pl.ds(m * CHUNK + g * L, L)]
                            anyp = ids16[0] == PAD_ID
                            for k in range(1, L):
                                anyp = jnp.logical_or(anyp,
                                                      ids16[k] == PAD_ID)

                            @pl.when(anyp)
                            def _():
                                for k in range(L):
                                    @pl.when(ids16[k] == PAD_ID)
                                    def _(k=k):
                                        f = (wid * cpw + m) * CHUNK + g * L + k
                                        blk = pl.multiple_of(
                                            jnp.bitwise_and(f, -8), 8)
                                        r = f - blk
                                        pltpu.sync_copy(
                                            out_hbm.at[pl.ds(blk, 8)],
                                            tok[0].at[pl.ds(0, 8)])
                                        w = lax.rem(f, SEQ)
                                        for j in range(HID // L):
                                            sl = pl.ds(j * L, L)
                                            tok[0][r, sl] = (tok[0][r, sl]
                                                             + pext[PAD_ROW, sl]
                                                             - pext[w, sl])
                                        pltpu.sync_copy(
                                            tok[0].at[pl.ds(0, 8)],
                                            out_hbm.at[pl.ds(blk, 8)])

    return pl.kernel(
        body,
        out_type=(jax.ShapeDtypeStruct((n_tok, HID), jnp.float32),
                  jax.ShapeDtypeStruct((n_tok, HID), jnp.float32)),
        mesh=mesh,
        scratch_types=[
            pltpu.VMEM((n_tok // NW,), jnp.int32),
            pltpu.VMEM((PEXT_ROWS, HID), jnp.float32),
            pltpu.VMEM((CHUNK, HID), jnp.float32),
            pltpu.VMEM((CHUNK, HID), jnp.float32),
            pltpu.VMEM((CHUNK, HID), jnp.float32),
            pltpu.VMEM((CHUNK, HID), jnp.float32),
            pltpu.SemaphoreType.DMA,
            pltpu.SemaphoreType.DMA,
            pltpu.SemaphoreType.DMA,
            pltpu.SemaphoreType.DMA,
            pltpu.SemaphoreType.DMA,
            pltpu.SemaphoreType.DMA,
            pltpu.SemaphoreType.DMA,
            pltpu.SemaphoreType.DMA,
        ],
    )


def kernel(enc_ids, dec_ids, src_table, trg_table, pos_table):
    B, T = enc_ids.shape
    n_tok = B * T
    enc_flat = enc_ids.astype(jnp.int32).reshape(n_tok)
    dec_flat = dec_ids.astype(jnp.int32).reshape(n_tok)
    wrap = jnp.concatenate([pos_table[1:SEQ + 1], pos_table[1:CHUNK + 1],
                            pos_table[0:1],
                            jnp.zeros((PEXT_ROWS - PAD_ROW - 1, HID),
                                      jnp.float32)])
    enc_o, dec_o = _build(n_tok)(enc_flat, dec_flat, src_table, trg_table,
                                 wrap)
    return enc_o.reshape(B, T, HID), dec_o.reshape(B, T, HID)


# speedup vs baseline: 1.0193x; 1.0193x over previous
import jax
import jax.numpy as jnp
from jax import lax
from jax.experimental import pallas as pl
from jax.experimental.pallas import tpu as pltpu
from jax.experimental.pallas import tpu_sc as plsc

NC = 2
NS = 16
L = 16
NW = NC * NS
CHUNK = 128
HID = 128
SEQ = 200
PAD_ID = 0
PAD_ROW = SEQ + CHUNK
PEXT_ROWS = 336


def _build(n_tok):
    per_w = n_tok // NW
    cpw = per_w // CHUNK
    assert cpw % 2 == 0
    mesh = plsc.VectorSubcoreMesh(core_axis_name="c", subcore_axis_name="s")

    def body(enc_ids, dec_ids, src_tab, trg_tab, pext_hbm, enc_out, dec_out,
             idx_big, pext, tok0, tok1, out0, out1,
             sem_t0, sem_t1, sem_o0, sem_o1):
        wid = lax.axis_index("s") * NC + lax.axis_index("c")
        tok = (tok0, tok1)
        out = (out0, out1)
        sem_t = (sem_t0, sem_t1)
        sem_o = (sem_o0, sem_o1)

        pltpu.sync_copy(pext_hbm, pext)

        for ids_hbm, tab_hbm, out_hbm in ((enc_ids, src_tab, enc_out),
                                          (dec_ids, trg_tab, dec_out)):
            pltpu.sync_copy(ids_hbm.at[pl.ds(wid * per_w, per_w)], idx_big)

            def issue(m, s):
                pltpu.async_copy(tab_hbm.at[idx_big.at[pl.ds(m * CHUNK, CHUNK)]],
                                 tok[s], sem_t[s])

            def wait_out(m, s):
                base = (wid * cpw + m) * CHUNK
                pltpu.make_async_copy(out[s], out_hbm.at[pl.ds(base, CHUNK)],
                                      sem_o[s]).wait()

            def consume(m, s, r0):
                pltpu.make_async_copy(tab_hbm.at[idx_big.at[pl.ds(m * CHUNK,
                                                                  CHUNK)]],
                                      tok[s], sem_t[s]).wait()
                base = (wid * cpw + m) * CHUNK

                @pl.when(m > 1)
                def _():
                    wait_out(m, s)

                @plsc.parallel_loop(0, CHUNK, unroll=4)
                def _tok(i):
                    i2 = r0 + i
                    for j in range(HID // L):
                        sl = pl.ds(j * L, L)
                        out[s][i, sl] = tok[s][i, sl] + pext[i2, sl]

                pltpu.async_copy(out[s], out_hbm.at[pl.ds(base, CHUNK)],
                                 sem_o[s])

            issue(0, 0)
            issue(1, 1)

            def r0_next(r0):
                r = r0 + CHUNK % SEQ
                return jnp.where(r >= SEQ, r - SEQ, r)

            @pl.loop(0, cpw, step=2, init_carry=jnp.int32(0))
            def _chunks(c, r0):
                consume(c, 0, r0)

                @pl.when(c + 2 < cpw)
                def _():
                    issue(c + 2, 0)

                r1 = r0_next(r0)
                consume(c + 1, 1, r1)

                @pl.when(c + 3 < cpw)
                def _():
                    issue(c + 3, 1)

                return r0_next(r1)

            for s in (0, 1):
                wait_out(0, s)

            @pl.loop(0, per_w // L, init_carry=jnp.zeros((L,), jnp.int32))
            def _gate(g, acc):
                ids16 = idx_big[pl.ds(g * L, L)]
                return acc + jnp.where(ids16 == PAD_ID, 1, 0)

            macc = _gate
            tot = macc[0]
            for k in range(1, L):
                tot = tot + macc[k]

            @pl.when(tot > 0)
            def _():
                @pl.loop(0, cpw)
                def _chk(m):
                    cacc = None
                    for g in range(CHUNK // L):
                        ids16 = idx_big[pl.ds(m * CHUNK + g * L, L)]
                        pm = jnp.where(ids16 == PAD_ID, 1, 0)
                        cacc = pm if cacc is None else cacc + pm
                    ctot = cacc[0]
                    for k in range(1, L):
                        ctot = ctot + cacc[k]

                    @pl.when(ctot > 0)
                    def _():
                        @pl.loop(0, CHUNK // L)
                        def _grp(g):
                            ids16 = idx_big[pl.ds(m * CHUNK + g * L, L)]
                            anyp = ids16[0] == PAD_ID
                            for k in range(1, L):
                                anyp = jnp.logical_or(anyp,
                                                      ids16[k] == PAD_ID)

                            @pl.when(anyp)
                            def _():
                                for k in range(L):
                                    @pl.when(ids16[k] == PAD_ID)
                                    def _(k=k):
                                        f = (wid * cpw + m) * CHUNK + g * L + k
                                        blk = pl.multiple_of(
                                            jnp.bitwise_and(f, -8), 8)
                                        r = f - blk
                                        pltpu.sync_copy(
                                            out_hbm.at[pl.ds(blk, 8)],
                                            tok[0].at[pl.ds(0, 8)])
                                        w = lax.rem(f, SEQ)
                                        for j in range(HID // L):
                                            sl = pl.ds(j * L, L)
                                            tok[0][r, sl] = (tok[0][r, sl]
                                                             + pext[PAD_ROW, sl]
                                                             - pext[w, sl])
                                        pltpu.sync_copy(
                                            tok[0].at[pl.ds(0, 8)],
                                            out_hbm.at[pl.ds(blk, 8)])

    return pl.kernel(
        body,
        out_type=(jax.ShapeDtypeStruct((n_tok, HID), jnp.float32),
                  jax.ShapeDtypeStruct((n_tok, HID), jnp.float32)),
        mesh=mesh,
        scratch_types=[
            pltpu.VMEM((n_tok // NW,), jnp.int32),
            pltpu.VMEM((PEXT_ROWS, HID), jnp.float32),
            pltpu.VMEM((CHUNK, HID), jnp.float32),
            pltpu.VMEM((CHUNK, HID), jnp.float32),
            pltpu.VMEM((CHUNK, HID), jnp.float32),
            pltpu.VMEM((CHUNK, HID), jnp.float32),
            pltpu.SemaphoreType.DMA,
            pltpu.SemaphoreType.DMA,
            pltpu.SemaphoreType.DMA,
            pltpu.SemaphoreType.DMA,
        ],
    )


def kernel(enc_ids, dec_ids, src_table, trg_table, pos_table):
    B, T = enc_ids.shape
    n_tok = B * T
    enc_flat = enc_ids.astype(jnp.int32).reshape(n_tok)
    dec_flat = dec_ids.astype(jnp.int32).reshape(n_tok)
    wrap = jnp.concatenate([pos_table[1:SEQ + 1], pos_table[1:CHUNK + 1],
                            pos_table[0:1],
                            jnp.zeros((PEXT_ROWS - PAD_ROW - 1, HID),
                                      jnp.float32)])
    enc_o, dec_o = _build(n_tok)(enc_flat, dec_flat, src_table, trg_table,
                                 wrap)
    return enc_o.reshape(B, T, HID), dec_o.reshape(B, T, HID)
